# Initial kernel scaffold; baseline (speedup 1.0000x reference)
import jax
import jax.numpy as jnp
from jax.experimental import pallas as pl


def kernel(xyz1, xyz2, points1, points2, W0, b0, g0, be0, W1, b1, g1, be1):
    raise NotImplementedError("write your pallas kernel here")


# speedup vs baseline: 16.1608x; 16.1608x over previous
import functools

import jax
import jax.numpy as jnp
from jax import lax
from jax.experimental import pallas as pl


_HI = lax.Precision.HIGHEST


def _dot(a, b):
    return lax.dot_general(a, b, (((1,), (0,)), ((), ())),
                           preferred_element_type=jnp.float32)


def _p1_body(x1_ref, x2_ref, p2_ref, p1_ref, w0_ref, b0_ref,
             z0_ref, sum_ref, sq_ref, *, TN, S):
    b = pl.program_id(0)
    nt = pl.program_id(1)

    @pl.when(jnp.logical_and(b == 0, nt == 0))
    def _():
        sum_ref[...] = jnp.zeros_like(sum_ref)
        sq_ref[...] = jnp.zeros_like(sq_ref)

    x1b = x1_ref[0]
    x2b = x2_ref[0]
    p2b = p2_ref[0]
    p1b = p1_ref[0]

    x1sq = x1b * x1b
    x2sq = x2b * x2b
    n2 = x2sq[0:1] + x2sq[1:2] + x2sq[2:3]
    ones3 = jnp.ones((3, 1), jnp.float32)
    n1 = lax.dot_general(x1sq, ones3, (((0,), (0,)), ((), ())),
                         preferred_element_type=jnp.float32, precision=_HI)
    prod = lax.dot_general(x1b, x2b, (((0,), (0,)), ((), ())),
                           preferred_element_type=jnp.float32)
    d = jnp.maximum(n1 + n2 - 2.0 * prod, 0.0)

    iota = lax.broadcasted_iota(jnp.int32, (TN, S), 1)
    wm = jnp.zeros((TN, S), jnp.float32)
    wsum = jnp.zeros((TN, 1), jnp.float32)
    for _ in range(3):
        m = jnp.min(d, axis=1, keepdims=True)
        sel = jnp.min(jnp.where(d == m, iota, S), axis=1, keepdims=True)
        onehot = iota == sel
        wk = 1.0 / (m + 1e-8)
        wm = wm + jnp.where(onehot, wk, 0.0)
        wsum = wsum + wk
        d = jnp.where(onehot, jnp.float32(jnp.inf), d)
    wm = wm / wsum

    interp_t = lax.dot_general(p2b, wm, (((1,), (1,)), ((), ())),
                               preferred_element_type=jnp.float32,
                               precision=_HI)
    w0 = w0_ref[...]
    z0 = _dot(w0[:, 0:64], p1b) + _dot(w0[:, 64:128], interp_t) + b0_ref[...]
    z0_ref[0] = z0
    sum_ref[...] += jnp.sum(z0, axis=1, keepdims=True)
    sq_ref[...] += jnp.sum(z0 * z0, axis=1, keepdims=True)


def _p2_body(z0_ref, sc_ref, sh_ref, w1_ref, b1_ref,
             z1_ref, sum_ref, sq_ref):
    b = pl.program_id(0)
    nt = pl.program_id(1)

    @pl.when(jnp.logical_and(b == 0, nt == 0))
    def _():
        sum_ref[...] = jnp.zeros_like(sum_ref)
        sq_ref[...] = jnp.zeros_like(sq_ref)

    y0 = jnp.maximum(z0_ref[0] * sc_ref[...] + sh_ref[...], 0.0)
    z1 = _dot(w1_ref[...], y0) + b1_ref[...]
    z1_ref[0] = z1
    sum_ref[...] += jnp.sum(z1, axis=1, keepdims=True)
    sq_ref[...] += jnp.sum(z1 * z1, axis=1, keepdims=True)


def _p3_body(z1_ref, sc_ref, sh_ref, out_ref):
    out_ref[0] = jnp.maximum(z1_ref[0] * sc_ref[...] + sh_ref[...], 0.0)


def kernel(xyz1, xyz2, points1, points2, W0, b0, g0, be0, W1, b1, g1, be1):
    B, _, N = xyz1.shape
    S = xyz2.shape[2]
    TN = 512
    NT = N // TN
    P = B * N

    col = lambda v: v.reshape(64, 1)

    f1 = jax.ShapeDtypeStruct((64, 1), jnp.float32)
    z0, s0, q0 = pl.pallas_call(
        functools.partial(_p1_body, TN=TN, S=S),
        grid=(B, NT),
        in_specs=[
            pl.BlockSpec((1, 3, TN), lambda b, n: (b, 0, n)),
            pl.BlockSpec((1, 3, S), lambda b, n: (b, 0, 0)),
            pl.BlockSpec((1, 64, S), lambda b, n: (b, 0, 0)),
            pl.BlockSpec((1, 64, TN), lambda b, n: (b, 0, n)),
            pl.BlockSpec((64, 128), lambda b, n: (0, 0)),
            pl.BlockSpec((64, 1), lambda b, n: (0, 0)),
        ],
        out_specs=[
            pl.BlockSpec((1, 64, TN), lambda b, n: (b, 0, n)),
            pl.BlockSpec((64, 1), lambda b, n: (0, 0)),
            pl.BlockSpec((64, 1), lambda b, n: (0, 0)),
        ],
        out_shape=[jax.ShapeDtypeStruct((B, 64, N), jnp.float32), f1, f1],
    )(xyz1, xyz2, points2, points1, W0, col(b0))

    mean0 = s0 / P
    var0 = q0 / P - mean0 * mean0
    sc0 = col(g0) / jnp.sqrt(var0 + 1e-5)
    sh0 = col(be0) - mean0 * sc0

    z1, s1, q1 = pl.pallas_call(
        _p2_body,
        grid=(B, NT),
        in_specs=[
            pl.BlockSpec((1, 64, TN), lambda b, n: (b, 0, n)),
            pl.BlockSpec((64, 1), lambda b, n: (0, 0)),
            pl.BlockSpec((64, 1), lambda b, n: (0, 0)),
            pl.BlockSpec((64, 64), lambda b, n: (0, 0)),
            pl.BlockSpec((64, 1), lambda b, n: (0, 0)),
        ],
        out_specs=[
            pl.BlockSpec((1, 64, TN), lambda b, n: (b, 0, n)),
            pl.BlockSpec((64, 1), lambda b, n: (0, 0)),
            pl.BlockSpec((64, 1), lambda b, n: (0, 0)),
        ],
        out_shape=[jax.ShapeDtypeStruct((B, 64, N), jnp.float32), f1, f1],
    )(z0, sc0, sh0, W1, col(b1))

    mean1 = s1 / P
    var1 = q1 / P - mean1 * mean1
    sc1 = col(g1) / jnp.sqrt(var1 + 1e-5)
    sh1 = col(be1) - mean1 * sc1

    out = pl.pallas_call(
        _p3_body,
        grid=(B, NT),
        in_specs=[
            pl.BlockSpec((1, 64, TN), lambda b, n: (b, 0, n)),
            pl.BlockSpec((64, 1), lambda b, n: (0, 0)),
            pl.BlockSpec((64, 1), lambda b, n: (0, 0)),
        ],
        out_specs=pl.BlockSpec((1, 64, TN), lambda b, n: (b, 0, n)),
        out_shape=jax.ShapeDtypeStruct((B, 64, N), jnp.float32),
    )(z1, sc1, sh1)

    return out


# speedup vs baseline: 19.9199x; 1.2326x over previous
import functools

import jax
import jax.numpy as jnp
from jax import lax
from jax.experimental import pallas as pl


_HI = lax.Precision.HIGHEST


def _dot(a, b, prec=None):
    return lax.dot_general(a, b, (((1,), (0,)), ((), ())),
                           preferred_element_type=jnp.float32, precision=prec)


def _p1_body(x1_ref, x2_ref, p2_ref, p1_ref, w0_ref, b0_ref,
             z0_ref, sum_ref, sq_ref, *, TN, S):
    b = pl.program_id(0)
    nt = pl.program_id(1)

    @pl.when(jnp.logical_and(b == 0, nt == 0))
    def _():
        sum_ref[...] = jnp.zeros_like(sum_ref)
        sq_ref[...] = jnp.zeros_like(sq_ref)

    x1b = x1_ref[0]
    x2b = x2_ref[0]
    p2b = p2_ref[0]
    p1b = p1_ref[0]

    x1sq = x1b * x1b
    x2sq = x2b * x2b
    n2 = x2sq[0:1] + x2sq[1:2] + x2sq[2:3]
    ones3 = jnp.ones((3, 1), jnp.float32)
    n1 = lax.dot_general(x1sq, ones3, (((0,), (0,)), ((), ())),
                         preferred_element_type=jnp.float32, precision=_HI)
    prod = lax.dot_general(x1b, x2b, (((0,), (0,)), ((), ())),
                           preferred_element_type=jnp.float32)
    d0 = jnp.maximum(n1 + n2 - 2.0 * prod, 0.0)

    iota = lax.broadcasted_iota(jnp.int32, (TN, S), 1)
    inf = jnp.float32(jnp.inf)
    d = d0
    wsum = jnp.zeros((TN, 1), jnp.float32)
    for _ in range(3):
        m = jnp.min(d, axis=1, keepdims=True)
        cand = jnp.where(d == m, iota, S)
        sel = jnp.min(cand, axis=1, keepdims=True)
        d = jnp.where(cand == sel, inf, d)
        wsum = wsum + 1.0 / (m + 1e-8)
    wm = jnp.where(d == d0, 0.0, 1.0 / (d0 + 1e-8)) * (1.0 / wsum)

    interp_t = lax.dot_general(p2b, wm, (((1,), (1,)), ((), ())),
                               preferred_element_type=jnp.float32,
                               precision=_HI)
    w0 = w0_ref[...]
    z0 = _dot(w0[:, 0:64], p1b) + _dot(w0[:, 64:128], interp_t) + b0_ref[...]
    z0_ref[0] = z0
    sum_ref[...] += jnp.sum(z0, axis=1, keepdims=True)
    sq_ref[...] += jnp.sum(z0 * z0, axis=1, keepdims=True)


def _bn_affine(s_ref, q_ref, g_ref, be_ref, P):
    mean = s_ref[...] / P
    var = q_ref[...] / P - mean * mean
    sc = g_ref[...] * lax.rsqrt(var + 1e-5)
    sh = be_ref[...] - mean * sc
    return sc, sh


def _p2_body(z0_ref, s0_ref, q0_ref, g0_ref, be0_ref, w1_ref, b1_ref,
             z1_ref, sum_ref, sq_ref, *, P):
    b = pl.program_id(0)
    nt = pl.program_id(1)

    @pl.when(jnp.logical_and(b == 0, nt == 0))
    def _():
        sum_ref[...] = jnp.zeros_like(sum_ref)
        sq_ref[...] = jnp.zeros_like(sq_ref)

    sc, sh = _bn_affine(s0_ref, q0_ref, g0_ref, be0_ref, P)
    y0 = jnp.maximum(z0_ref[0] * sc + sh, 0.0)
    z1 = _dot(w1_ref[...], y0) + b1_ref[...]
    z1_ref[0] = z1
    sum_ref[...] += jnp.sum(z1, axis=1, keepdims=True)
    sq_ref[...] += jnp.sum(z1 * z1, axis=1, keepdims=True)


def _p3_body(z1_ref, s1_ref, q1_ref, g1_ref, be1_ref, out_ref, *, P):
    sc, sh = _bn_affine(s1_ref, q1_ref, g1_ref, be1_ref, P)
    out_ref[0] = jnp.maximum(z1_ref[0] * sc + sh, 0.0)


def _c64(i_map):
    return pl.BlockSpec((64, 1), i_map)


def kernel(xyz1, xyz2, points1, points2, W0, b0, g0, be0, W1, b1, g1, be1):
    B, _, N = xyz1.shape
    S = xyz2.shape[2]
    TN = 1024
    NT = N // TN
    P = float(B * N)

    col = lambda v: v.reshape(64, 1)
    c0 = lambda b, n: (0, 0)

    f1 = jax.ShapeDtypeStruct((64, 1), jnp.float32)
    z0, s0, q0 = pl.pallas_call(
        functools.partial(_p1_body, TN=TN, S=S),
        grid=(B, NT),
        in_specs=[
            pl.BlockSpec((1, 3, TN), lambda b, n: (b, 0, n)),
            pl.BlockSpec((1, 3, S), lambda b, n: (b, 0, 0)),
            pl.BlockSpec((1, 64, S), lambda b, n: (b, 0, 0)),
            pl.BlockSpec((1, 64, TN), lambda b, n: (b, 0, n)),
            pl.BlockSpec((64, 128), c0),
            _c64(c0),
        ],
        out_specs=[
            pl.BlockSpec((1, 64, TN), lambda b, n: (b, 0, n)),
            _c64(c0),
            _c64(c0),
        ],
        out_shape=[jax.ShapeDtypeStruct((B, 64, N), jnp.float32), f1, f1],
    )(xyz1, xyz2, points2, points1, W0, col(b0))

    z1, s1, q1 = pl.pallas_call(
        functools.partial(_p2_body, P=P),
        grid=(B, NT),
        in_specs=[
            pl.BlockSpec((1, 64, TN), lambda b, n: (b, 0, n)),
            _c64(c0), _c64(c0), _c64(c0), _c64(c0),
            pl.BlockSpec((64, 64), c0),
            _c64(c0),
        ],
        out_specs=[
            pl.BlockSpec((1, 64, TN), lambda b, n: (b, 0, n)),
            _c64(c0),
            _c64(c0),
        ],
        out_shape=[jax.ShapeDtypeStruct((B, 64, N), jnp.float32), f1, f1],
    )(z0, s0, q0, col(g0), col(be0), W1, col(b1))

    out = pl.pallas_call(
        functools.partial(_p3_body, P=P),
        grid=(B, NT),
        in_specs=[
            pl.BlockSpec((1, 64, TN), lambda b, n: (b, 0, n)),
            _c64(c0), _c64(c0), _c64(c0), _c64(c0),
        ],
        out_specs=pl.BlockSpec((1, 64, TN), lambda b, n: (b, 0, n)),
        out_shape=jax.ShapeDtypeStruct((B, 64, N), jnp.float32),
    )(z1, s1, q1, col(g1), col(be1))

    return out
